# baseline (speedup 1.0000x reference)
import functools
import math

import jax
import jax.numpy as jnp
from jax import lax
from jax.experimental import pallas as pl
from jax.experimental.pallas import tpu as pltpu
from jax.experimental.pallas import tpu_sc as plsc

N, C, K, H, FF = 16384, 256, 27, 8, 512
DH = C // H
NW = 32
CH = 128
NCHUNK = (N * K) // (NW * CH)
BN = 128


def _sc_gather(table, idx2):
    mesh = plsc.VectorSubcoreMesh(core_axis_name="c", subcore_axis_name="s")

    @functools.partial(
        pl.kernel,
        out_type=jax.ShapeDtypeStruct((N * K, C), jnp.float32),
        mesh=mesh,
        scratch_types=[
            pltpu.VMEM((NCHUNK, CH), jnp.int32),
            pltpu.VMEM((CH, C), jnp.float32),
            pltpu.VMEM((CH, C), jnp.float32),
            pltpu.SemaphoreType.DMA,
            pltpu.SemaphoreType.DMA,
        ],
    )
    def gather_kernel(table_hbm, idx_hbm, out_hbm, idx_v, buf0, buf1, sem0, sem1):
        wid = lax.axis_index("s") * 2 + lax.axis_index("c")
        base = wid * NCHUNK
        pltpu.sync_copy(idx_hbm.at[wid], idx_v)
        pltpu.make_async_copy(table_hbm.at[idx_v.at[0]], buf0, sem0).start()
        pltpu.make_async_copy(table_hbm.at[idx_v.at[1]], buf1, sem1).start()

        def step(t, carry):
            j = t * 2
            for b, (buf, sem) in enumerate(((buf0, sem0), (buf1, sem1))):
                jj = j + b
                pltpu.make_async_copy(table_hbm.at[idx_v.at[jj]], buf, sem).wait()
                pltpu.sync_copy(buf, out_hbm.at[pl.ds((base + jj) * CH, CH)])

                @pl.when(jj + 2 < NCHUNK)
                def _():
                    pltpu.make_async_copy(
                        table_hbm.at[idx_v.at[jj + 2]], buf, sem
                    ).start()

            return carry

        lax.fori_loop(0, NCHUNK // 2, step, 0)

    return gather_kernel(table, idx2)


def _tc_body(vf_ref, co_ref, g_ref, mk_ref, wq_ref, wk_ref, wv_ref, wo_ref,
             w1_ref, w2_ref, pw_ref, pv_ref, pb1_ref, out_ref):
    f32 = jnp.float32
    dims_tt = (((1,), (1,)), ((), ()))
    pv = pv_ref[...]

    vf = vf_ref[...]
    q = lax.dot_general(vf, wq_ref[...], dims_tt, preferred_element_type=f32)
    q = q + pv[0:1, :]
    wk = wk_ref[...]
    qts = []
    for h in range(H):
        qts.append(
            lax.dot_general(q[:, h * DH:(h + 1) * DH], wk[h * DH:(h + 1) * DH, :],
                            (((1,), (0,)), ((), ())), preferred_element_type=f32))
    qt = jnp.stack(qts, axis=1)

    pw = pw_ref[...]
    pb = pv[8:9, :]
    feats = []
    sks = []
    for k in range(K):
        posk = (co_ref[:, 0, k:k + 1] * pw[0:1, :]
                + co_ref[:, 1, k:k + 1] * pw[1:2, :]
                + co_ref[:, 2, k:k + 1] * pw[2:3, :] + pb)
        fk = g_ref[:, k * C:(k + 1) * C] + jnp.maximum(posk, 0.0)
        feats.append(fk)
        sks.append(jnp.sum(qt * fk[:, None, :], axis=-1))

    scores = jnp.stack(sks, axis=-1) * (1.0 / math.sqrt(DH))
    mk = mk_ref[...]
    scores = jnp.where(mk[:, None, :] > 0.5, -1e9, scores)
    m = jnp.max(scores, axis=-1, keepdims=True)
    e = jnp.exp(scores - m)
    attn = e / jnp.sum(e, axis=-1, keepdims=True)

    mixed = attn[:, :, 0:1] * feats[0][:, None, :]
    for k in range(1, K):
        mixed = mixed + attn[:, :, k:k + 1] * feats[k][:, None, :]

    wv = wv_ref[...]
    vhs = []
    for h in range(H):
        vhs.append(
            lax.dot_general(mixed[:, h, :], wv[h * DH:(h + 1) * DH, :], dims_tt,
                            preferred_element_type=f32))
    att = jnp.concatenate(vhs, axis=1) + pv[1:2, :]
    att = lax.dot_general(att, wo_ref[...], dims_tt,
                          preferred_element_type=f32) + pv[2:3, :]

    def ln(x, g_row, b_row):
        mu = jnp.mean(x, axis=-1, keepdims=True)
        d = x - mu
        var = jnp.mean(d * d, axis=-1, keepdims=True)
        return d * lax.rsqrt(var + 1e-5) * g_row + b_row

    x = ln(vf + att, pv[4:5, :], pv[5:6, :])
    ff = jnp.maximum(
        lax.dot_general(x, w1_ref[...], dims_tt, preferred_element_type=f32)
        + pb1_ref[0:1, :], 0.0)
    f2 = lax.dot_general(ff, w2_ref[...], dims_tt,
                         preferred_element_type=f32) + pv[3:4, :]
    out_ref[...] = ln(x + f2, pv[6:7, :], pv[7:8, :])


def kernel(voxel_features, key_coords, Wq, Wk, Wv, bq, bk, bv, Wo, bo,
           W1, b1, W2, b2, ln1_g, ln1_b, ln2_g, ln2_b, pos_w, pos_b,
           key_indices, key_mask):
    del bk

    idx2 = key_indices.astype(jnp.int32).reshape(NW, NCHUNK, CH)
    gathered = _sc_gather(voxel_features, idx2).reshape(N, K * C)

    maskf = key_mask.astype(jnp.float32)
    zc = jnp.zeros((1, C), jnp.float32)
    pv = jnp.concatenate(
        [bq[None, :], bv[None, :], bo[None, :], b2[None, :],
         ln1_g[None, :], ln1_b[None, :], ln2_g[None, :], ln2_b[None, :],
         pos_b[None, :], zc, zc, zc, zc, zc, zc, zc], axis=0)
    pw = jnp.concatenate([pos_w.T, jnp.zeros((5, C), jnp.float32)], axis=0)
    pb1 = jnp.broadcast_to(b1[None, :], (8, FF))

    grid = N // BN
    full = lambda shape: pl.BlockSpec(shape, lambda i: tuple(0 for _ in shape))
    out = pl.pallas_call(
        _tc_body,
        grid=(grid,),
        in_specs=[
            pl.BlockSpec((BN, C), lambda i: (i, 0)),
            pl.BlockSpec((BN, 3, K), lambda i: (i, 0, 0)),
            pl.BlockSpec((BN, K * C), lambda i: (i, 0)),
            pl.BlockSpec((BN, K), lambda i: (i, 0)),
            full((C, C)), full((C, C)), full((C, C)), full((C, C)),
            full((FF, C)), full((C, FF)),
            full((8, C)), full((16, C)), full((8, FF)),
        ],
        out_specs=pl.BlockSpec((BN, C), lambda i: (i, 0)),
        out_shape=jax.ShapeDtypeStruct((N, C), jnp.float32),
        compiler_params=pltpu.CompilerParams(
            dimension_semantics=("arbitrary",)),
    )(voxel_features, key_coords, gathered, maskf, Wq, Wk, Wv, Wo,
      W1, W2, pw, pv, pb1)
    return out


# speedup vs baseline: 4.4710x; 4.4710x over previous
import functools
import math

import jax
import jax.numpy as jnp
from jax import lax
from jax.experimental import pallas as pl
from jax.experimental.pallas import tpu as pltpu
from jax.experimental.pallas import tpu_sc as plsc

N, C, K, H, FF = 16384, 256, 27, 8, 512
DH = C // H
NW = 32
CH = 128
NCHUNK = (N * K) // (NW * CH)
BN = 128
KB = K * BN


def _sc_gather(table, idx2):
    mesh = plsc.VectorSubcoreMesh(core_axis_name="c", subcore_axis_name="s")

    @functools.partial(
        pl.kernel,
        out_type=jax.ShapeDtypeStruct((N * K, C), jnp.float32),
        mesh=mesh,
        scratch_types=[
            pltpu.VMEM((NCHUNK, CH), jnp.int32),
            pltpu.VMEM((CH, C), jnp.float32),
            pltpu.VMEM((CH, C), jnp.float32),
            pltpu.SemaphoreType.DMA,
            pltpu.SemaphoreType.DMA,
        ],
    )
    def gather_kernel(table_hbm, idx_hbm, out_hbm, idx_v, buf0, buf1, sem0, sem1):
        wid = lax.axis_index("s") * 2 + lax.axis_index("c")
        base = wid * NCHUNK
        pltpu.sync_copy(idx_hbm.at[wid], idx_v)
        pltpu.make_async_copy(table_hbm.at[idx_v.at[0]], buf0, sem0).start()
        pltpu.make_async_copy(table_hbm.at[idx_v.at[1]], buf1, sem1).start()

        def step(t, carry):
            j = t * 2
            for b, (buf, sem) in enumerate(((buf0, sem0), (buf1, sem1))):
                jj = j + b
                pltpu.make_async_copy(table_hbm.at[idx_v.at[jj]], buf, sem).wait()
                pltpu.sync_copy(buf, out_hbm.at[pl.ds((base + jj) * CH, CH)])

                @pl.when(jj + 2 < NCHUNK)
                def _():
                    pltpu.make_async_copy(
                        table_hbm.at[idx_v.at[jj + 2]], buf, sem
                    ).start()

            return carry

        lax.fori_loop(0, NCHUNK // 2, step, 0)

    return gather_kernel(table, idx2)


def _tc_body(vf_ref, co_ref, g_ref, mk_ref, wq_ref, wk_ref, wv_ref, wo_ref,
             w1_ref, w2_ref, pw_ref, pv_ref, pb1_ref, seg_ref, out_ref):
    f32 = jnp.float32
    bf16 = jnp.bfloat16
    dims_tt = (((1,), (1,)), ((), ()))
    dims_nn = (((1,), (0,)), ((), ()))
    pv = pv_ref[...]
    seg = seg_ref[...]

    vf = vf_ref[...]
    q = lax.dot_general(vf, wq_ref[...], dims_tt, preferred_element_type=f32)
    q = (q + pv[0:1, :]) * (1.0 / math.sqrt(DH))

    coo = co_ref[...].reshape(KB, 3)
    pos = lax.dot_general(coo, pw_ref[...], dims_tt, preferred_element_type=f32)
    feats = g_ref[...].reshape(KB, C) + jnp.maximum(pos + pv[8:9, :], 0.0)

    fb = feats.astype(bf16)
    kp = lax.dot_general(fb, wk_ref[...], dims_tt, preferred_element_type=f32)
    vp = lax.dot_general(fb, wv_ref[...], dims_tt, preferred_element_type=f32)

    qk = jnp.broadcast_to(q[None, :, :], (K, BN, C)).reshape(KB, C)
    s = lax.dot_general(qk * kp, seg, dims_nn, preferred_element_type=f32)
    s3 = s.reshape(K, BN, H)
    s3 = jnp.where(mk_ref[...][:, :, None] > 0.5, -1e9, s3)
    m = jnp.max(s3, axis=0, keepdims=True)
    e = jnp.exp(s3 - m)
    attn = (e / jnp.sum(e, axis=0, keepdims=True)).reshape(KB, H)

    ab = lax.dot_general(attn, seg, (((1,), (1,)), ((), ())),
                         preferred_element_type=f32)
    mixed = jnp.sum((ab * vp).reshape(K, BN, C), axis=0)

    att = lax.dot_general(mixed + pv[1:2, :], wo_ref[...], dims_tt,
                          preferred_element_type=f32) + pv[2:3, :]

    def ln(x, g_row, b_row):
        mu = jnp.mean(x, axis=-1, keepdims=True)
        d = x - mu
        var = jnp.mean(d * d, axis=-1, keepdims=True)
        return d * lax.rsqrt(var + 1e-5) * g_row + b_row

    x = ln(vf + att, pv[4:5, :], pv[5:6, :])
    ff = jnp.maximum(
        lax.dot_general(x, w1_ref[...], dims_tt, preferred_element_type=f32)
        + pb1_ref[0:1, :], 0.0)
    f2 = lax.dot_general(ff, w2_ref[...], dims_tt,
                         preferred_element_type=f32) + pv[3:4, :]
    out_ref[...] = ln(x + f2, pv[6:7, :], pv[7:8, :])


def kernel(voxel_features, key_coords, Wq, Wk, Wv, bq, bk, bv, Wo, bo,
           W1, b1, W2, b2, ln1_g, ln1_b, ln2_g, ln2_b, pos_w, pos_b,
           key_indices, key_mask):
    del bk

    idx_t = key_indices.T.astype(jnp.int32).reshape(NW, NCHUNK, CH)
    gathered = _sc_gather(voxel_features, idx_t).reshape(K, N, C)

    coords_t = key_coords.transpose(2, 0, 1)
    maskf = key_mask.T.astype(jnp.float32)
    zc = jnp.zeros((1, C), jnp.float32)
    pv = jnp.concatenate(
        [bq[None, :], bv[None, :], bo[None, :], b2[None, :],
         ln1_g[None, :], ln1_b[None, :], ln2_g[None, :], ln2_b[None, :],
         pos_b[None, :], zc, zc, zc, zc, zc, zc, zc], axis=0)
    pb1 = jnp.broadcast_to(b1[None, :], (8, FF))
    seg = (jnp.arange(C)[:, None] // DH ==
           jnp.arange(H)[None, :]).astype(jnp.float32)
    wkb = Wk.astype(jnp.bfloat16)
    wvb = Wv.astype(jnp.bfloat16)

    grid = N // BN
    full = lambda shape: pl.BlockSpec(shape, lambda i: tuple(0 for _ in shape))
    out = pl.pallas_call(
        _tc_body,
        grid=(grid,),
        in_specs=[
            pl.BlockSpec((BN, C), lambda i: (i, 0)),
            pl.BlockSpec((K, BN, 3), lambda i: (0, i, 0)),
            pl.BlockSpec((K, BN, C), lambda i: (0, i, 0)),
            pl.BlockSpec((K, BN), lambda i: (0, i)),
            full((C, C)), full((C, C)), full((C, C)), full((C, C)),
            full((FF, C)), full((C, FF)),
            full((C, 3)), full((16, C)), full((8, FF)), full((C, H)),
        ],
        out_specs=pl.BlockSpec((BN, C), lambda i: (i, 0)),
        out_shape=jax.ShapeDtypeStruct((N, C), jnp.float32),
        compiler_params=pltpu.CompilerParams(
            dimension_semantics=("arbitrary",)),
    )(voxel_features, coords_t, gathered, maskf, Wq, wkb, wvb, Wo,
      W1, W2, pos_w, pv, pb1, seg)
    return out
